# Initial kernel scaffold; baseline (speedup 1.0000x reference)
import jax
import jax.numpy as jnp
from jax.experimental import pallas as pl


def kernel(node_embeddings, adjacency_lists, W1, b1, W2, b2):
    raise NotImplementedError("write your pallas kernel here")


# speedup vs baseline: 16.6037x; 16.6037x over previous
import functools

import jax
import jax.numpy as jnp
from jax import lax
from jax.experimental import pallas as pl
from jax.experimental.pallas import tpu as pltpu
from jax.experimental.pallas import tpu_sc as plsc

NP_PAD = 10240
NW = 32
EB = 128
ROWS_PER_TILE = NP_PAD // 16

_MESH = plsc.VectorSubcoreMesh(core_axis_name="c", subcore_axis_name="s")


def _sc_degree(dst_blocks):
    nb = dst_blocks.shape[1]

    @functools.partial(
        pl.kernel,
        out_type=jax.ShapeDtypeStruct((2, NP_PAD, 16), jnp.float32),
        mesh=_MESH,
        scratch_types=[
            pltpu.VMEM((nb, EB), jnp.int32),
            pltpu.VMEM((EB, 16), jnp.float32),
            pltpu.VMEM((16, 16), jnp.float32),
            pltpu.VMEM_SHARED((NP_PAD, 16), jnp.float32),
        ],
    )
    def deg_kernel(dst_hbm, out_hbm, dst_v, ones_v, zb_v, acc):
        c = lax.axis_index("c")
        s = lax.axis_index("s")
        wid = c * 16 + s
        row0 = s * ROWS_PER_TILE
        zeros16 = jnp.zeros((16,), jnp.float32)
        ones16 = jnp.ones((16,), jnp.float32)
        for r in range(16):
            zb_v[r, :] = zeros16
        for r in range(EB):
            ones_v[r, :] = ones16

        @pl.loop(0, ROWS_PER_TILE // 16)
        def _(i):
            pltpu.sync_copy(zb_v, acc.at[pl.ds(row0 + i * 16, 16), :])

        pltpu.sync_copy(dst_hbm.at[wid], dst_v)
        plsc.subcore_barrier()

        @pl.loop(0, nb)
        def _(j):
            pltpu.sync_copy(ones_v, acc.at[dst_v.at[j]], add=True)

        plsc.subcore_barrier()

        @pl.loop(0, ROWS_PER_TILE // EB)
        def _(k):
            r = row0 + k * EB
            pltpu.sync_copy(acc.at[pl.ds(r, EB), :], ones_v)
            pltpu.sync_copy(ones_v, out_hbm.at[c, pl.ds(r, EB), :])

    return deg_kernel(dst_blocks)


def _sc_spmm(table, src_blocks, dst_blocks, feat):
    nb = src_blocks.shape[1]
    params = None
    if feat % 128 != 0:
        params = pltpu.CompilerParams(use_tc_tiling_on_sc=False)

    @functools.partial(
        pl.kernel,
        out_type=jax.ShapeDtypeStruct((2, NP_PAD, feat), jnp.float32),
        mesh=_MESH,
        compiler_params=params,
        scratch_types=[
            pltpu.VMEM((nb, EB), jnp.int32),
            pltpu.VMEM((nb, EB), jnp.int32),
            pltpu.VMEM((EB, feat), jnp.float32),
            pltpu.VMEM((16, feat), jnp.float32),
            pltpu.VMEM_SHARED((NP_PAD, feat), jnp.float32),
            pltpu.SemaphoreType.DMA,
        ],
    )
    def spmm_kernel(tab_hbm, src_hbm, dst_hbm, out_hbm,
                    src_v, dst_v, buf, zb_v, acc, sem):
        c = lax.axis_index("c")
        s = lax.axis_index("s")
        wid = c * 16 + s
        row0 = s * ROWS_PER_TILE
        zeros16 = jnp.zeros((16,), jnp.float32)
        for r in range(16):
            for k in range(feat // 16):
                zb_v[r, pl.ds(k * 16, 16)] = zeros16

        @pl.loop(0, ROWS_PER_TILE // 16)
        def _(i):
            pltpu.sync_copy(zb_v, acc.at[pl.ds(row0 + i * 16, 16), :])

        pltpu.sync_copy(src_hbm.at[wid], src_v)
        pltpu.sync_copy(dst_hbm.at[wid], dst_v)
        plsc.subcore_barrier()

        @pl.loop(0, nb)
        def _(j):
            pltpu.async_copy(tab_hbm.at[src_v.at[j]], buf, sem).wait()
            pltpu.sync_copy(buf, acc.at[dst_v.at[j]], add=True)

        plsc.subcore_barrier()

        @pl.loop(0, ROWS_PER_TILE // EB)
        def _(k):
            r = row0 + k * EB
            pltpu.sync_copy(acc.at[pl.ds(r, EB), :], buf)
            pltpu.sync_copy(buf, out_hbm.at[c, pl.ds(r, EB), :])

    return spmm_kernel(table, src_blocks, dst_blocks)


def _tc_scale(x_p, W1, dp0, dp1):
    rb = 1024
    d = x_p.shape[1]
    h = W1.shape[1]

    def body(x_ref, w_ref, d0_ref, d1_ref, xs_ref, a_ref):
        deg = d0_ref[...] + d1_ref[...] + 1.0
        a = lax.rsqrt(deg)
        xw = jnp.dot(x_ref[...], w_ref[...], preferred_element_type=jnp.float32)
        xs_ref[...] = xw * a
        a_ref[...] = a

    return pl.pallas_call(
        body,
        grid=(NP_PAD // rb,),
        in_specs=[
            pl.BlockSpec((rb, d), lambda i: (i, 0)),
            pl.BlockSpec((d, h), lambda i: (0, 0)),
            pl.BlockSpec((rb, 1), lambda i: (i, 0)),
            pl.BlockSpec((rb, 1), lambda i: (i, 0)),
        ],
        out_specs=[
            pl.BlockSpec((rb, h), lambda i: (i, 0)),
            pl.BlockSpec((rb, 1), lambda i: (i, 0)),
        ],
        out_shape=[
            jax.ShapeDtypeStruct((NP_PAD, h), jnp.float32),
            jax.ShapeDtypeStruct((NP_PAD, 1), jnp.float32),
        ],
    )(x_p, W1, dp0, dp1)


def _tc_hidden(p0, p1, x1s, a, b1, W2):
    rb = 1024
    h = x1s.shape[1]
    co = W2.shape[1]

    def body(p0_ref, p1_ref, xs_ref, a_ref, b1_ref, w2_ref, out_ref):
        agg = a_ref[...] * (p0_ref[...] + p1_ref[...] + xs_ref[...]) + b1_ref[...]
        hid = jnp.maximum(agg, 0.0)
        hw = jnp.dot(hid, w2_ref[...], preferred_element_type=jnp.float32)
        out_ref[...] = a_ref[...] * hw

    return pl.pallas_call(
        body,
        grid=(NP_PAD // rb,),
        in_specs=[
            pl.BlockSpec((rb, h), lambda i: (i, 0)),
            pl.BlockSpec((rb, h), lambda i: (i, 0)),
            pl.BlockSpec((rb, h), lambda i: (i, 0)),
            pl.BlockSpec((rb, 1), lambda i: (i, 0)),
            pl.BlockSpec((1, h), lambda i: (0, 0)),
            pl.BlockSpec((h, co), lambda i: (0, 0)),
        ],
        out_specs=pl.BlockSpec((rb, co), lambda i: (i, 0)),
        out_shape=jax.ShapeDtypeStruct((NP_PAD, co), jnp.float32),
    )(p0, p1, x1s, a, b1, W2)


def _tc_softmax(q0, q1, h2s, a, b2):
    rb = 1024
    co = h2s.shape[1]

    def body(q0_ref, q1_ref, h_ref, a_ref, b2_ref, out_ref):
        z = a_ref[...] * (q0_ref[...] + q1_ref[...] + h_ref[...]) + b2_ref[...]
        z = z - jnp.max(z, axis=1, keepdims=True)
        e = jnp.exp(z)
        out_ref[...] = e / jnp.sum(e, axis=1, keepdims=True)

    return pl.pallas_call(
        body,
        grid=(NP_PAD // rb,),
        in_specs=[
            pl.BlockSpec((rb, co), lambda i: (i, 0)),
            pl.BlockSpec((rb, co), lambda i: (i, 0)),
            pl.BlockSpec((rb, co), lambda i: (i, 0)),
            pl.BlockSpec((rb, 1), lambda i: (i, 0)),
            pl.BlockSpec((1, co), lambda i: (0, 0)),
        ],
        out_specs=pl.BlockSpec((rb, co), lambda i: (i, 0)),
        out_shape=jax.ShapeDtypeStruct((NP_PAD, co), jnp.float32),
    )(q0, q1, h2s, a, b2)


def kernel(node_embeddings, adjacency_lists, W1, b1, W2, b2):
    n, d = node_embeddings.shape
    e = adjacency_lists.shape[1]
    src = adjacency_lists[0].astype(jnp.int32)
    dst = adjacency_lists[1].astype(jnp.int32)

    nb = -(-e // (NW * EB))
    ep = NW * nb * EB
    src_p = jnp.concatenate(
        [src, jnp.zeros((ep - e,), jnp.int32)]).reshape(NW, nb, EB)
    dst_p = jnp.concatenate(
        [dst, jnp.full((ep - e,), n, jnp.int32)]).reshape(NW, nb, EB)
    x_p = jnp.pad(node_embeddings, ((0, NP_PAD - n), (0, 0)))

    degp = _sc_degree(dst_p)
    dp0 = degp[0, :, :1]
    dp1 = degp[1, :, :1]
    x1s, a = _tc_scale(x_p, W1, dp0, dp1)
    agg1 = _sc_spmm(x1s, src_p, dst_p, W1.shape[1])
    h2s = _tc_hidden(agg1[0], agg1[1], x1s, a, b1.reshape(1, -1), W2)
    agg2 = _sc_spmm(h2s, src_p, dst_p, W2.shape[1])
    probs = _tc_softmax(agg2[0], agg2[1], h2s, a, b2.reshape(1, -1))
    return probs[:n]
